# baseline (speedup 1.0000x reference)
import functools

import jax
import jax.numpy as jnp
from jax import lax
from jax.experimental import pallas as pl
from jax.experimental.pallas import tpu as pltpu
from jax.experimental.pallas import tpu_sc as plsc

N = 10000
E = 320000
N_PAD = 10240
D_IN = 128
N_CLASSES = 40


def _pad2(w, rows, cols):
    return jnp.pad(w, ((0, rows - w.shape[0]), (0, cols - w.shape[1])))


def _pad1(b, n, value=0.0):
    return jnp.pad(b, (0, n - b.shape[0]), constant_values=value)


def _node_mm_body(h_ref, wk_ref, bk_ref, wqv_ref, bqv_ref, ws_ref, k_ref,
                  qv_ref, s_ref):
    h = h_ref[...]
    k_ref[...] = jnp.dot(h, wk_ref[...], preferred_element_type=jnp.float32) + bk_ref[...]
    qv_ref[...] = jnp.dot(h, wqv_ref[...], preferred_element_type=jnp.float32) + bqv_ref[...]
    s_ref[...] = jnp.dot(h, ws_ref[...], preferred_element_type=jnp.float32)


def _node_mm(h, wk, bk, wqv, bqv, ws, F):
    npad, cin = h.shape
    BR = 2560
    grid = npad // BR
    return pl.pallas_call(
        _node_mm_body,
        grid=(grid,),
        in_specs=[
            pl.BlockSpec((BR, cin), lambda i: (i, 0)),
            pl.BlockSpec((cin, F), lambda i: (0, 0)),
            pl.BlockSpec((1, F), lambda i: (0, 0)),
            pl.BlockSpec((cin, 2 * F), lambda i: (0, 0)),
            pl.BlockSpec((1, 2 * F), lambda i: (0, 0)),
            pl.BlockSpec((cin, F), lambda i: (0, 0)),
        ],
        out_specs=[
            pl.BlockSpec((BR, F), lambda i: (i, 0)),
            pl.BlockSpec((BR, 2 * F), lambda i: (i, 0)),
            pl.BlockSpec((BR, F), lambda i: (i, 0)),
        ],
        out_shape=[
            jax.ShapeDtypeStruct((npad, F), jnp.float32),
            jax.ShapeDtypeStruct((npad, 2 * F), jnp.float32),
            jax.ShapeDtypeStruct((npad, F), jnp.float32),
        ],
    )(h, wk, bk.reshape(1, F), wqv, bqv.reshape(1, 2 * F), ws)


def _edge_mm_body(a_ref, w_ref, b_ref, o_ref):
    o_ref[...] = jnp.dot(a_ref[...], w_ref[0], preferred_element_type=jnp.float32) + b_ref[0]


def _edge_mm(edge_attr, we, be, F):
    e_rows, ed = edge_attr.shape
    F2 = F // 2
    BE = 8000
    grid = e_rows // BE
    we_split = jnp.stack([we[:, :F2], we[:, F2:]])
    be_split = jnp.stack([be[:F2].reshape(1, F2), be[F2:].reshape(1, F2)])
    return pl.pallas_call(
        _edge_mm_body,
        grid=(2, grid),
        in_specs=[
            pl.BlockSpec((BE, ed), lambda h, i: (i, 0)),
            pl.BlockSpec((1, ed, F2), lambda h, i: (h, 0, 0)),
            pl.BlockSpec((1, 1, F2), lambda h, i: (h, 0, 0)),
        ],
        out_specs=pl.BlockSpec((BE, F2), lambda h, i: (h * grid + i, 0)),
        out_shape=jax.ShapeDtypeStruct((2 * e_rows, F2), jnp.float32),
    )(edge_attr, we_split, be_split)


def _post_body(agg_ref, s_ref, b_ref, gamma_ref, beta_ref, o_ref):
    agg = jnp.concatenate([agg_ref[0, :N, :], agg_ref[1, :N, :]], axis=1)
    h = agg + s_ref[:N, :] + b_ref[...]
    h = jnp.where(h >= 0.0, h, 0.01 * h)
    mean = jnp.mean(h, axis=0, keepdims=True)
    var = jnp.mean((h - mean) * (h - mean), axis=0, keepdims=True)
    o_ref[...] = gamma_ref[...] * (h - mean) * lax.rsqrt(var + 1e-5) + beta_ref[...]


def _post(agg_pair, s, b, gamma, beta, F):
    return pl.pallas_call(
        _post_body,
        out_shape=jax.ShapeDtypeStruct((N, F), jnp.float32),
    )(agg_pair, s, b.reshape(1, F), gamma.reshape(1, F), beta.reshape(1, F))


_NC, _NS = 2, 16
_NW = _NC * _NS


@functools.lru_cache(maxsize=None)
def _make_edge_sc(F):
    F2 = F // 2
    EPW = E // _NS
    C = 80
    NCHUNK = EPW // C
    RPT = N_PAD // _NS

    mesh = plsc.VectorSubcoreMesh(core_axis_name="c", subcore_axis_name="s",
                                  num_cores=_NC, num_subcores=_NS)

    def body(k_hbm, qv_hbm, e_hbm, src_hbm, dst_hbm, out_hbm,
             srcv, dstv, srcov, dstov, kdv, qvv, ev, msgv, aggsh,
             sem_k, sem_qv, sem_e):
        c = lax.axis_index("c")
        s = lax.axis_index("s")
        roff = c * N_PAD

        def zero_row(r, carry):
            for j in range(F2 // 16):
                msgv[r, pl.ds(j * 16, 16)] = jnp.zeros((16,), jnp.float32)
            return carry
        lax.fori_loop(0, C, zero_row, 0)
        for t in range(RPT // C):
            pltpu.sync_copy(msgv, aggsh.at[pl.ds(s * RPT + t * C, C)])
        plsc.subcore_barrier()

        def chunk(ci, carry):
            base = s * EPW + ci * C
            pltpu.sync_copy(src_hbm.at[pl.ds(base, C)], srcv)
            pltpu.sync_copy(dst_hbm.at[pl.ds(base, C)], dstv)
            for j in range(C // 16):
                sl = pl.ds(j * 16, 16)
                srcov[sl] = srcv[sl] + roff
                dstov[sl] = dstv[sl] + roff
            cp_k = pltpu.async_copy(k_hbm.at[dstov], kdv, sem_k)
            cp_qv = pltpu.async_copy(qv_hbm.at[srcov], qvv, sem_qv)
            cp_e = pltpu.async_copy(e_hbm.at[pl.ds(c * E + base, C)], ev, sem_e)
            cp_k.wait()
            cp_qv.wait()
            cp_e.wait()

            def edge(r, carry2):
                for j in range(F2 // 16):
                    sl = pl.ds(j * 16, 16)
                    kd = kdv[r, sl]
                    qj = qvv[r, sl]
                    vj = qvv[r, pl.ds(F2 + j * 16, 16)]
                    ee = ev[r, sl]
                    z = kd + qj + ee + ee
                    gate = 1.0 / (1.0 + jnp.exp(-z))
                    msgv[r, sl] = gate * (vj + ee)
                return carry2
            lax.fori_loop(0, C, edge, 0)
            pltpu.sync_copy(msgv, aggsh.at[dstv], add=True)
            return carry
        lax.fori_loop(0, NCHUNK, chunk, 0)
        plsc.subcore_barrier()
        pltpu.sync_copy(aggsh.at[pl.ds(s * RPT, RPT)],
                        out_hbm.at[c, pl.ds(s * RPT, RPT)])

    return pl.kernel(
        body,
        out_type=jax.ShapeDtypeStruct((_NC, N_PAD, F2), jnp.float32),
        mesh=mesh,
        compiler_params=pltpu.CompilerParams(use_tc_tiling_on_sc=False),
        scratch_types=[
            pltpu.VMEM((C,), jnp.int32),
            pltpu.VMEM((C,), jnp.int32),
            pltpu.VMEM((C,), jnp.int32),
            pltpu.VMEM((C,), jnp.int32),
            pltpu.VMEM((C, F2), jnp.float32),
            pltpu.VMEM((C, F), jnp.float32),
            pltpu.VMEM((C, F2), jnp.float32),
            pltpu.VMEM((C, F2), jnp.float32),
            pltpu.VMEM_SHARED((N_PAD, F2), jnp.float32),
            pltpu.SemaphoreType.DMA,
            pltpu.SemaphoreType.DMA,
            pltpu.SemaphoreType.DMA,
        ],
    )


def _edge_stage(k, qv, e_split, src, dst, F):
    F2 = F // 2
    k_split = jnp.concatenate([k[:, :F2], k[:, F2:]], axis=0)
    q, v = qv[:, :F], qv[:, F:]
    qv_split = jnp.concatenate(
        [jnp.concatenate([q[:, :F2], v[:, :F2]], axis=1),
         jnp.concatenate([q[:, F2:], v[:, F2:]], axis=1)], axis=0)
    return _make_edge_sc(F)(k_split, qv_split, e_split, src, dst)


def _layer(h_pad, edge_attr, src, dst, p, nrm, F):
    cin = h_pad.shape[1]
    wk = _pad2(p["Wk"], cin, F)
    bk = _pad1(p["bk"], F)
    wqv = jnp.concatenate([_pad2(p["Wq"], cin, F), _pad2(p["Wv"], cin, F)], axis=1)
    bqv = jnp.concatenate([_pad1(p["bq"], F), _pad1(p["bv"], F)])
    ws = _pad2(p["Ws"], cin, F)
    we = _pad2(p["We"], p["We"].shape[0], F)
    be = _pad1(p["be"], F)
    b = _pad1(p["b"], F)
    gamma = _pad1(nrm["gamma"], F, value=1.0)
    beta = _pad1(nrm["beta"], F)

    k, qv, s = _node_mm(h_pad, wk, bk, wqv, bqv, ws, F)
    e = _edge_mm(edge_attr, we, be, F)
    agg_pair = _edge_stage(k, qv, e, src, dst, F)
    return _post(agg_pair, s, b, gamma, beta, F)


def kernel(x, edge_index, batch, edge_attr, params):
    src = edge_index[0]
    dst = edge_index[1]
    h = x
    for i, F in ((1, 128), (2, 128), (3, 128)):
        h_pad = jnp.pad(h, ((0, N_PAD - N), (0, 0)))
        h = _layer(h_pad, edge_attr, src, dst, params["conv%d" % i],
                   params["norm%d" % i], F)
    return h[:, :N_CLASSES]


# speedup vs baseline: 3.4981x; 3.4981x over previous
import functools

import jax
import jax.numpy as jnp
from jax import lax
from jax.experimental import pallas as pl
from jax.experimental.pallas import tpu as pltpu
from jax.experimental.pallas import tpu_sc as plsc

N = 10000
E = 320000
N_PAD = 10240
D_IN = 128
N_CLASSES = 40


def _pad2(w, rows, cols):
    return jnp.pad(w, ((0, rows - w.shape[0]), (0, cols - w.shape[1])))


def _pad1(b, n, value=0.0):
    return jnp.pad(b, (0, n - b.shape[0]), constant_values=value)


def _node_mm_body(h_ref, wk_ref, bk_ref, wqv_ref, bqv_ref, ws_ref, k_ref,
                  qv_ref, s_ref):
    h = h_ref[...]
    k_ref[...] = jnp.dot(h, wk_ref[...], preferred_element_type=jnp.float32) + bk_ref[...]
    qv_ref[...] = jnp.dot(h, wqv_ref[...], preferred_element_type=jnp.float32) + bqv_ref[...]
    s_ref[...] = jnp.dot(h, ws_ref[...], preferred_element_type=jnp.float32)


def _node_mm(h, wk, bk, wqv, bqv, ws, F):
    npad, cin = h.shape
    BR = 2560
    grid = npad // BR
    return pl.pallas_call(
        _node_mm_body,
        grid=(grid,),
        in_specs=[
            pl.BlockSpec((BR, cin), lambda i: (i, 0)),
            pl.BlockSpec((cin, F), lambda i: (0, 0)),
            pl.BlockSpec((1, F), lambda i: (0, 0)),
            pl.BlockSpec((cin, 2 * F), lambda i: (0, 0)),
            pl.BlockSpec((1, 2 * F), lambda i: (0, 0)),
            pl.BlockSpec((cin, F), lambda i: (0, 0)),
        ],
        out_specs=[
            pl.BlockSpec((BR, F), lambda i: (i, 0)),
            pl.BlockSpec((BR, 2 * F), lambda i: (i, 0)),
            pl.BlockSpec((BR, F), lambda i: (i, 0)),
        ],
        out_shape=[
            jax.ShapeDtypeStruct((npad, F), jnp.float32),
            jax.ShapeDtypeStruct((npad, 2 * F), jnp.float32),
            jax.ShapeDtypeStruct((npad, F), jnp.float32),
        ],
    )(h, wk, bk.reshape(1, F), wqv, bqv.reshape(1, 2 * F), ws)


def _edge_mm_body(a_ref, w_ref, b_ref, o_ref):
    o_ref[...] = jnp.dot(a_ref[...], w_ref[0], preferred_element_type=jnp.float32) + b_ref[0]


def _edge_mm(edge_attr, we, be, F):
    e_rows, ed = edge_attr.shape
    F2 = F // 2
    BE = 8000
    grid = e_rows // BE
    we_split = jnp.stack([we[:, :F2], we[:, F2:]])
    be_split = jnp.stack([be[:F2].reshape(1, F2), be[F2:].reshape(1, F2)])
    return pl.pallas_call(
        _edge_mm_body,
        grid=(2, grid),
        in_specs=[
            pl.BlockSpec((BE, ed), lambda h, i: (i, 0)),
            pl.BlockSpec((1, ed, F2), lambda h, i: (h, 0, 0)),
            pl.BlockSpec((1, 1, F2), lambda h, i: (h, 0, 0)),
        ],
        out_specs=pl.BlockSpec((BE, F2), lambda h, i: (h * grid + i, 0)),
        out_shape=jax.ShapeDtypeStruct((2 * e_rows, F2), jnp.float32),
    )(edge_attr, we_split, be_split)


def _post_body(agg_ref, s_ref, b_ref, gamma_ref, beta_ref, o_ref):
    agg = jnp.concatenate([agg_ref[0, :N, :], agg_ref[1, :N, :]], axis=1)
    h = agg + s_ref[:N, :] + b_ref[...]
    h = jnp.where(h >= 0.0, h, 0.01 * h)
    mean = jnp.mean(h, axis=0, keepdims=True)
    var = jnp.mean((h - mean) * (h - mean), axis=0, keepdims=True)
    o_ref[...] = gamma_ref[...] * (h - mean) * lax.rsqrt(var + 1e-5) + beta_ref[...]


def _post(agg_pair, s, b, gamma, beta, F):
    return pl.pallas_call(
        _post_body,
        out_shape=jax.ShapeDtypeStruct((N, F), jnp.float32),
    )(agg_pair, s, b.reshape(1, F), gamma.reshape(1, F), beta.reshape(1, F))


_NC, _NS = 2, 16
_NW = _NC * _NS


@functools.lru_cache(maxsize=None)
def _make_edge_sc(F):
    F2 = F // 2
    EPW = E // _NS
    C = 80
    NCHUNK = EPW // C
    RPT = N_PAD // _NS

    mesh = plsc.VectorSubcoreMesh(core_axis_name="c", subcore_axis_name="s",
                                  num_cores=_NC, num_subcores=_NS)

    def body(k_hbm, qv_hbm, e_hbm, src_hbm, dst_hbm, out_hbm,
             sraw0, sraw1, draw0, draw1, sgat0, sgat1, dgat0, dgat1,
             dsts0, dsts1,
             kdv0, kdv1, qvv0, qvv1, ev0, ev1, msgv0, msgv1, aggsh,
             sem_i0, sem_i1, sem_g0, sem_g1, sem_s0, sem_s1):
        c = lax.axis_index("c")
        s = lax.axis_index("s")
        roff = c * N_PAD
        sraw = (sraw0, sraw1)
        draw = (draw0, draw1)
        sgat = (sgat0, sgat1)
        dgat = (dgat0, dgat1)
        dsts = (dsts0, dsts1)
        kdv = (kdv0, kdv1)
        qvv = (qvv0, qvv1)
        ev = (ev0, ev1)
        msgv = (msgv0, msgv1)
        sem_i = (sem_i0, sem_i1)
        sem_g = (sem_g0, sem_g1)
        sem_s = (sem_s0, sem_s1)
        tbase = s * EPW

        def fire_idx(b, ci):
            pltpu.async_copy(src_hbm.at[pl.ds(tbase + ci * C, C)], sraw[b],
                             sem_i[b])
            pltpu.async_copy(dst_hbm.at[pl.ds(tbase + ci * C, C)], draw[b],
                             sem_i[b])

        def wait_idx(b):
            pltpu.make_async_copy(src_hbm.at[pl.ds(0, C)], sraw[b],
                                  sem_i[b]).wait()
            pltpu.make_async_copy(dst_hbm.at[pl.ds(0, C)], draw[b],
                                  sem_i[b]).wait()

        def fill(b):
            for j in range(C // 16):
                sl = pl.ds(j * 16, 16)
                dgat[b][sl] = draw[b][sl] + roff
                sgat[b][sl] = sraw[b][sl] + roff

        def fire_gathers(b, ci):
            pltpu.async_copy(k_hbm.at[dgat[b]], kdv[b], sem_g[b])
            pltpu.async_copy(qv_hbm.at[sgat[b]], qvv[b], sem_g[b])
            pltpu.async_copy(e_hbm.at[pl.ds(c * E + tbase + ci * C, C)],
                             ev[b], sem_g[b])

        def wait_gathers(b):
            pltpu.make_async_copy(k_hbm.at[dgat[b]], kdv[b], sem_g[b]).wait()
            pltpu.make_async_copy(qv_hbm.at[sgat[b]], qvv[b], sem_g[b]).wait()
            pltpu.make_async_copy(e_hbm.at[pl.ds(0, C)], ev[b], sem_g[b]).wait()

        def wait_scatter(b):
            pltpu.make_async_copy(msgv[b], aggsh.at[dsts[b]],
                                  sem_s[b]).wait()

        fire_idx(0, 0)
        fire_idx(1, 1)
        wait_idx(0)
        fill(0)
        fire_gathers(0, 0)
        fire_idx(0, 2)
        wait_idx(1)
        fill(1)
        fire_gathers(1, 1)
        fire_idx(1, 3)

        def zero_row(r, carry):
            for j in range(F2 // 16):
                msgv0[r, pl.ds(j * 16, 16)] = jnp.zeros((16,), jnp.float32)
            return carry
        lax.fori_loop(0, C, zero_row, 0)
        for t in range(RPT // C):
            pltpu.sync_copy(msgv0, aggsh.at[pl.ds(s * RPT + t * C, C)])
        plsc.subcore_barrier()

        def super_chunk(g, carry):
            for b in range(2):
                ci = 2 * g + b
                wait_gathers(b)

                @pl.when(ci >= 2)
                def _():
                    wait_scatter(b)

                @plsc.parallel_loop(0, C, step=1, unroll=2)
                def edge(r):
                    for j in range(F2 // 16):
                        sl = pl.ds(j * 16, 16)
                        kd = kdv[b][r, sl]
                        qj = qvv[b][r, sl]
                        vj = qvv[b][r, pl.ds(F2 + j * 16, 16)]
                        ee = ev[b][r, sl]
                        z = kd + qj + ee + ee
                        gate = 1.0 / (1.0 + jnp.exp(-z))
                        msgv[b][r, sl] = gate * (vj + ee)

                for j in range(C // 16):
                    sl = pl.ds(j * 16, 16)
                    dsts[b][sl] = dgat[b][sl] - roff
                pltpu.async_copy(msgv[b], aggsh.at[dsts[b]], sem_s[b],
                                 add=True)

                @pl.when(ci + 2 < NCHUNK)
                def _():
                    wait_idx(b)
                    fill(b)
                    fire_gathers(b, ci + 2)

                    @pl.when(ci + 4 < NCHUNK)
                    def _():
                        fire_idx(b, ci + 4)
            return carry
        lax.fori_loop(0, NCHUNK // 2, super_chunk, 0)
        wait_scatter(0)
        wait_scatter(1)
        plsc.subcore_barrier()
        pltpu.sync_copy(aggsh.at[pl.ds(s * RPT, RPT)],
                        out_hbm.at[c, pl.ds(s * RPT, RPT)])

    return pl.kernel(
        body,
        out_type=jax.ShapeDtypeStruct((_NC, N_PAD, F2), jnp.float32),
        mesh=mesh,
        compiler_params=pltpu.CompilerParams(use_tc_tiling_on_sc=False),
        scratch_types=(
            [pltpu.VMEM((C,), jnp.int32) for _ in range(10)]
            + [
                pltpu.VMEM((C, F2), jnp.float32),
                pltpu.VMEM((C, F2), jnp.float32),
                pltpu.VMEM((C, F), jnp.float32),
                pltpu.VMEM((C, F), jnp.float32),
                pltpu.VMEM((C, F2), jnp.float32),
                pltpu.VMEM((C, F2), jnp.float32),
                pltpu.VMEM((C, F2), jnp.float32),
                pltpu.VMEM((C, F2), jnp.float32),
                pltpu.VMEM_SHARED((N_PAD, F2), jnp.float32),
            ]
            + [pltpu.SemaphoreType.DMA for _ in range(6)]
        ),
    )


def _edge_stage(k, qv, e_split, src, dst, F):
    F2 = F // 2
    k_split = jnp.concatenate([k[:, :F2], k[:, F2:]], axis=0)
    q, v = qv[:, :F], qv[:, F:]
    qv_split = jnp.concatenate(
        [jnp.concatenate([q[:, :F2], v[:, :F2]], axis=1),
         jnp.concatenate([q[:, F2:], v[:, F2:]], axis=1)], axis=0)
    return _make_edge_sc(F)(k_split, qv_split, e_split, src, dst)


def _layer(h_pad, edge_attr, src, dst, p, nrm, F):
    cin = h_pad.shape[1]
    wk = _pad2(p["Wk"], cin, F)
    bk = _pad1(p["bk"], F)
    wqv = jnp.concatenate([_pad2(p["Wq"], cin, F), _pad2(p["Wv"], cin, F)], axis=1)
    bqv = jnp.concatenate([_pad1(p["bq"], F), _pad1(p["bv"], F)])
    ws = _pad2(p["Ws"], cin, F)
    we = _pad2(p["We"], p["We"].shape[0], F)
    be = _pad1(p["be"], F)
    b = _pad1(p["b"], F)
    gamma = _pad1(nrm["gamma"], F, value=1.0)
    beta = _pad1(nrm["beta"], F)

    k, qv, s = _node_mm(h_pad, wk, bk, wqv, bqv, ws, F)
    e = _edge_mm(edge_attr, we, be, F)
    agg_pair = _edge_stage(k, qv, e, src, dst, F)
    return _post(agg_pair, s, b, gamma, beta, F)


def kernel(x, edge_index, batch, edge_attr, params):
    src = edge_index[0]
    dst = edge_index[1]
    h = x
    for i, F in ((1, 128), (2, 128), (3, 128)):
        h_pad = jnp.pad(h, ((0, N_PAD - N), (0, 0)))
        h = _layer(h_pad, edge_attr, src, dst, params["conv%d" % i],
                   params["norm%d" % i], F)
    return h[:, :N_CLASSES]


# speedup vs baseline: 5.2576x; 1.5030x over previous
import functools

import jax
import jax.numpy as jnp
from jax import lax
from jax.experimental import pallas as pl
from jax.experimental.pallas import tpu as pltpu
from jax.experimental.pallas import tpu_sc as plsc

N = 10000
E = 320000
N_PAD = 10240
D_IN = 128
N_CLASSES = 40


def _pad2(w, rows, cols):
    return jnp.pad(w, ((0, rows - w.shape[0]), (0, cols - w.shape[1])))


def _pad1(b, n, value=0.0):
    return jnp.pad(b, (0, n - b.shape[0]), constant_values=value)


def _node_mm_body(h_ref, wk_ref, bk_ref, wqv_ref, bqv_ref, ws_ref, k_ref,
                  qv_ref, s_ref):
    h = h_ref[...]
    k_ref[...] = jnp.dot(h, wk_ref[...], preferred_element_type=jnp.float32) + bk_ref[...]
    qv_ref[...] = jnp.dot(h, wqv_ref[...], preferred_element_type=jnp.float32) + bqv_ref[...]
    s_ref[...] = jnp.dot(h, ws_ref[...], preferred_element_type=jnp.float32)


def _node_mm(h, wk, bk, wqv, bqv, ws, F):
    npad, cin = h.shape
    BR = 2560
    grid = npad // BR
    return pl.pallas_call(
        _node_mm_body,
        grid=(grid,),
        in_specs=[
            pl.BlockSpec((BR, cin), lambda i: (i, 0)),
            pl.BlockSpec((cin, F), lambda i: (0, 0)),
            pl.BlockSpec((1, F), lambda i: (0, 0)),
            pl.BlockSpec((cin, 2 * F), lambda i: (0, 0)),
            pl.BlockSpec((1, 2 * F), lambda i: (0, 0)),
            pl.BlockSpec((cin, F), lambda i: (0, 0)),
        ],
        out_specs=[
            pl.BlockSpec((BR, F), lambda i: (i, 0)),
            pl.BlockSpec((BR, 2 * F), lambda i: (i, 0)),
            pl.BlockSpec((BR, F), lambda i: (i, 0)),
        ],
        out_shape=[
            jax.ShapeDtypeStruct((npad, F), jnp.float32),
            jax.ShapeDtypeStruct((npad, 2 * F), jnp.float32),
            jax.ShapeDtypeStruct((npad, F), jnp.float32),
        ],
    )(h, wk, bk.reshape(1, F), wqv, bqv.reshape(1, 2 * F), ws)


def _edge_mm_body(a_ref, w_ref, b_ref, o_ref):
    o_ref[...] = jnp.dot(a_ref[...], w_ref[...], preferred_element_type=jnp.float32) + b_ref[...]


def _edge_mm(edge_attr, we, be, F):
    e_rows, ed = edge_attr.shape
    BE = 8000
    grid = e_rows // BE
    return pl.pallas_call(
        _edge_mm_body,
        grid=(grid,),
        in_specs=[
            pl.BlockSpec((BE, ed), lambda i: (i, 0)),
            pl.BlockSpec((ed, F), lambda i: (0, 0)),
            pl.BlockSpec((1, F), lambda i: (0, 0)),
        ],
        out_specs=pl.BlockSpec((BE, F), lambda i: (i, 0)),
        out_shape=jax.ShapeDtypeStruct((e_rows, F), jnp.float32),
    )(edge_attr, we, be.reshape(1, F))


def _post_body(agg_ref, s_ref, b_ref, gamma_ref, beta_ref, o_ref):
    agg = jnp.concatenate([agg_ref[0, :N, :], agg_ref[1, :N, :]], axis=1)
    h = agg + s_ref[:N, :] + b_ref[...]
    h = jnp.where(h >= 0.0, h, 0.01 * h)
    mean = jnp.mean(h, axis=0, keepdims=True)
    var = jnp.mean((h - mean) * (h - mean), axis=0, keepdims=True)
    o_ref[...] = gamma_ref[...] * (h - mean) * lax.rsqrt(var + 1e-5) + beta_ref[...]


def _post(agg_pair, s, b, gamma, beta, F):
    return pl.pallas_call(
        _post_body,
        out_shape=jax.ShapeDtypeStruct((N, F), jnp.float32),
    )(agg_pair, s, b.reshape(1, F), gamma.reshape(1, F), beta.reshape(1, F))


_NC, _NS = 2, 16
_NW = _NC * _NS


@functools.lru_cache(maxsize=None)
def _make_edge_sc(F):
    F2 = F // 2
    EPW = E // _NS
    C = 80
    NCHUNK = EPW // C
    RPT = N_PAD // _NS

    mesh = plsc.VectorSubcoreMesh(core_axis_name="c", subcore_axis_name="s",
                                  num_cores=_NC, num_subcores=_NS)

    def body(k_hbm, qv_hbm, e_hbm, src_hbm, dst_hbm, out_hbm,
             sraw0, sraw1, draw0, draw1, sgat0, sgat1, dgat0, dgat1,
             dsts0, dsts1,
             kdv0, kdv1, qvv0, qvv1, ev0, ev1, msgv0, msgv1, aggsh,
             sem_i0, sem_i1, sem_g0, sem_g1, sem_s0, sem_s1):
        c = lax.axis_index("c")
        s = lax.axis_index("s")
        roff = c * N_PAD
        sraw = (sraw0, sraw1)
        draw = (draw0, draw1)
        sgat = (sgat0, sgat1)
        dgat = (dgat0, dgat1)
        dsts = (dsts0, dsts1)
        kdv = (kdv0, kdv1)
        qvv = (qvv0, qvv1)
        ev = (ev0, ev1)
        msgv = (msgv0, msgv1)
        sem_i = (sem_i0, sem_i1)
        sem_g = (sem_g0, sem_g1)
        sem_s = (sem_s0, sem_s1)
        tbase = s * EPW

        def fire_idx(b, ci):
            pltpu.async_copy(src_hbm.at[pl.ds(tbase + ci * C, C)], sraw[b],
                             sem_i[b])
            pltpu.async_copy(dst_hbm.at[pl.ds(tbase + ci * C, C)], draw[b],
                             sem_i[b])

        def wait_idx(b):
            pltpu.make_async_copy(src_hbm.at[pl.ds(0, C)], sraw[b],
                                  sem_i[b]).wait()
            pltpu.make_async_copy(dst_hbm.at[pl.ds(0, C)], draw[b],
                                  sem_i[b]).wait()

        def fill(b):
            for j in range(C // 16):
                sl = pl.ds(j * 16, 16)
                dgat[b][sl] = draw[b][sl] + roff
                sgat[b][sl] = sraw[b][sl] + roff

        def fire_gathers(b, ci):
            pltpu.async_copy(k_hbm.at[dgat[b]], kdv[b], sem_g[b])
            pltpu.async_copy(qv_hbm.at[sgat[b]], qvv[b], sem_g[b])
            pltpu.async_copy(
                e_hbm.at[pl.ds(tbase + ci * C, C), pl.ds(c * F2, F2)],
                ev[b], sem_g[b])

        def wait_gathers(b):
            pltpu.make_async_copy(k_hbm.at[dgat[b]], kdv[b], sem_g[b]).wait()
            pltpu.make_async_copy(qv_hbm.at[sgat[b]], qvv[b], sem_g[b]).wait()
            pltpu.make_async_copy(e_hbm.at[pl.ds(0, C), pl.ds(0, F2)],
                                  ev[b], sem_g[b]).wait()

        def wait_scatter(b):
            pltpu.make_async_copy(msgv[b], aggsh.at[dsts[b]],
                                  sem_s[b]).wait()

        fire_idx(0, 0)
        fire_idx(1, 1)
        wait_idx(0)
        fill(0)
        fire_gathers(0, 0)
        fire_idx(0, 2)
        wait_idx(1)
        fill(1)
        fire_gathers(1, 1)
        fire_idx(1, 3)

        def zero_row(r, carry):
            for j in range(F2 // 16):
                msgv0[r, pl.ds(j * 16, 16)] = jnp.zeros((16,), jnp.float32)
            return carry
        lax.fori_loop(0, C, zero_row, 0)
        for t in range(RPT // C):
            pltpu.sync_copy(msgv0, aggsh.at[pl.ds(s * RPT + t * C, C)])
        plsc.subcore_barrier()

        def super_chunk(g, carry):
            for b in range(2):
                ci = 2 * g + b
                wait_gathers(b)

                @pl.when(ci >= 2)
                def _():
                    wait_scatter(b)

                @plsc.parallel_loop(0, C, step=1, unroll=2)
                def edge(r):
                    for j in range(F2 // 16):
                        sl = pl.ds(j * 16, 16)
                        kd = kdv[b][r, sl]
                        qj = qvv[b][r, sl]
                        vj = qvv[b][r, pl.ds(F2 + j * 16, 16)]
                        ee = ev[b][r, sl]
                        z = kd + qj + ee + ee
                        gate = 1.0 / (1.0 + jnp.exp(-z))
                        msgv[b][r, sl] = gate * (vj + ee)

                for j in range(C // 16):
                    sl = pl.ds(j * 16, 16)
                    dsts[b][sl] = dgat[b][sl] - roff
                pltpu.async_copy(msgv[b], aggsh.at[dsts[b]], sem_s[b],
                                 add=True)

                @pl.when(ci + 2 < NCHUNK)
                def _():
                    wait_idx(b)
                    fill(b)
                    fire_gathers(b, ci + 2)

                    @pl.when(ci + 4 < NCHUNK)
                    def _():
                        fire_idx(b, ci + 4)
            return carry
        lax.fori_loop(0, NCHUNK // 2, super_chunk, 0)
        wait_scatter(0)
        wait_scatter(1)
        plsc.subcore_barrier()
        pltpu.sync_copy(aggsh.at[pl.ds(s * RPT, RPT)],
                        out_hbm.at[c, pl.ds(s * RPT, RPT)])

    return pl.kernel(
        body,
        out_type=jax.ShapeDtypeStruct((_NC, N_PAD, F2), jnp.float32),
        mesh=mesh,
        compiler_params=pltpu.CompilerParams(use_tc_tiling_on_sc=False),
        scratch_types=(
            [pltpu.VMEM((C,), jnp.int32) for _ in range(10)]
            + [
                pltpu.VMEM((C, F2), jnp.float32),
                pltpu.VMEM((C, F2), jnp.float32),
                pltpu.VMEM((C, F), jnp.float32),
                pltpu.VMEM((C, F), jnp.float32),
                pltpu.VMEM((C, F2), jnp.float32),
                pltpu.VMEM((C, F2), jnp.float32),
                pltpu.VMEM((C, F2), jnp.float32),
                pltpu.VMEM((C, F2), jnp.float32),
                pltpu.VMEM_SHARED((N_PAD, F2), jnp.float32),
            ]
            + [pltpu.SemaphoreType.DMA for _ in range(6)]
        ),
    )


def _edge_stage(k, qv, e_split, src, dst, F):
    F2 = F // 2
    k_split = jnp.concatenate([k[:, :F2], k[:, F2:]], axis=0)
    q, v = qv[:, :F], qv[:, F:]
    qv_split = jnp.concatenate(
        [jnp.concatenate([q[:, :F2], v[:, :F2]], axis=1),
         jnp.concatenate([q[:, F2:], v[:, F2:]], axis=1)], axis=0)
    return _make_edge_sc(F)(k_split, qv_split, e_split, src, dst)


def _layer(h_pad, edge_attr, src, dst, p, nrm, F):
    cin = h_pad.shape[1]
    wk = _pad2(p["Wk"], cin, F)
    bk = _pad1(p["bk"], F)
    wqv = jnp.concatenate([_pad2(p["Wq"], cin, F), _pad2(p["Wv"], cin, F)], axis=1)
    bqv = jnp.concatenate([_pad1(p["bq"], F), _pad1(p["bv"], F)])
    ws = _pad2(p["Ws"], cin, F)
    we = _pad2(p["We"], p["We"].shape[0], F)
    be = _pad1(p["be"], F)
    b = _pad1(p["b"], F)
    gamma = _pad1(nrm["gamma"], F, value=1.0)
    beta = _pad1(nrm["beta"], F)

    k, qv, s = _node_mm(h_pad, wk, bk, wqv, bqv, ws, F)
    e = _edge_mm(edge_attr, we, be, F)
    agg_pair = _edge_stage(k, qv, e, src, dst, F)
    return _post(agg_pair, s, b, gamma, beta, F)


def kernel(x, edge_index, batch, edge_attr, params):
    src = edge_index[0]
    dst = edge_index[1]
    h = x
    for i, F in ((1, 128), (2, 128), (3, 128)):
        h_pad = jnp.pad(h, ((0, N_PAD - N), (0, 0)))
        h = _layer(h_pad, edge_attr, src, dst, params["conv%d" % i],
                   params["norm%d" % i], F)
    return h[:, :N_CLASSES]
